# baseline (speedup 1.0000x reference)
import functools

import jax
import jax.numpy as jnp
from jax.experimental import pallas as pl
from jax.experimental.pallas import tpu as pltpu

_K = 8
_TAU = 0.7
_T = 256


def _gating_kernel(x_ref, w_ref, disp_ref, comb_ref, loss_ref, carry_ref,
                   *, cap, nblocks, G):
    i = pl.program_id(0)

    @pl.when(i == 0)
    def _init():
        carry_ref[...] = jnp.zeros_like(carry_ref)

    x = x_ref[0]
    w = w_ref[...]
    gates = jnp.dot(x, w, preferred_element_type=jnp.float32)

    m = jnp.max(gates, axis=-1, keepdims=True)
    ex = jnp.exp(gates - m)
    probs = ex / jnp.sum(ex, axis=-1, keepdims=True)

    colid = jax.lax.broadcasted_iota(jnp.int32, (_T, G), 1)

    work = probs
    csum = jnp.zeros((_T, 1), jnp.float32)
    onehots = jnp.zeros((_T, G), jnp.float32)
    vmap_raw = jnp.zeros((_T, G), jnp.float32)
    renorm = jnp.zeros((_T, 1), jnp.float32)
    for k in range(_K):
        mk = jnp.max(work, axis=-1, keepdims=True)
        ismax = work == mk
        idx = jnp.min(jnp.where(ismax, colid, G), axis=-1, keepdims=True)
        oh = (colid == idx).astype(jnp.float32)
        work = jnp.where(oh > 0.5, -1.0, work)
        csum = csum + mk
        if k == 0:
            keep = jnp.ones((_T, 1), jnp.float32)
        else:
            keep = (csum < _TAU).astype(jnp.float32)
        onehots = onehots + oh * keep
        vmap_raw = vmap_raw + oh * (mk * keep)
        renorm = renorm + mk * keep
    v_map = vmap_raw / jnp.maximum(renorm, 1e-7)

    rowid_t = jax.lax.broadcasted_iota(jnp.int32, (_T, _T), 0)
    colid_t = jax.lax.broadcasted_iota(jnp.int32, (_T, _T), 1)
    lt = (rowid_t > colid_t).astype(jnp.float32)
    excl = jnp.dot(lt, onehots, preferred_element_type=jnp.float32)
    rank = excl + carry_ref[...]

    carry_ref[...] += jnp.sum(onehots, axis=0, keepdims=True)

    placed = (onehots > 0.5) & (rank < cap - 0.5)
    r_map = jnp.where(placed, rank, -1.0)

    ci = jax.lax.broadcasted_iota(jnp.int32, (_T, G, cap), 2)
    r_i = r_map.astype(jnp.int32)
    disp = (ci == r_i[:, :, None]).astype(jnp.float32)
    disp_ref[0] = disp
    comb_ref[0] = v_map[:, :, None] * disp

    @pl.when(i == nblocks - 1)
    def _loss():
        usage = jnp.minimum(carry_ref[...], float(cap))
        mu = jnp.mean(usage)
        l = jnp.mean((usage - mu) ** 2) / (mu + 1e-8)
        loss_ref[0, 0] = jnp.where(jnp.sum(usage) > 0, l, 0.0)


def kernel(x, W):
    b, n, d = x.shape
    G = W.shape[1]
    cap = max(min(n, int(n * 1.25 / G)), 4)
    BN = b * n
    nblocks = BN // _T
    npb = n // _T
    disp, comb, loss = pl.pallas_call(
        functools.partial(_gating_kernel, cap=cap, nblocks=nblocks, G=G),
        grid=(nblocks,),
        in_specs=[
            pl.BlockSpec((1, _T, d), lambda i: (i // (n // _T), i % (n // _T), 0)),
            pl.BlockSpec((d, G), lambda i: (0, 0)),
        ],
        out_specs=[
            pl.BlockSpec((1, _T, G, cap),
                         lambda i: (i // (n // _T), i % (n // _T), 0, 0)),
            pl.BlockSpec((1, _T, G, cap),
                         lambda i: (i // (n // _T), i % (n // _T), 0, 0)),
            pl.BlockSpec(memory_space=pltpu.SMEM),
        ],
        out_shape=[
            jax.ShapeDtypeStruct((b, n, G, cap), jnp.float32),
            jax.ShapeDtypeStruct((b, n, G, cap), jnp.float32),
            jax.ShapeDtypeStruct((1, 1), jnp.float32),
        ],
        scratch_shapes=[pltpu.VMEM((1, G), jnp.float32)],
        compiler_params=pltpu.CompilerParams(
            dimension_semantics=("arbitrary",)),
    )(x, W)
    return (disp, comb, loss.reshape(()))


# speedup vs baseline: 7.2582x; 7.2582x over previous
import functools

import jax
import jax.numpy as jnp
from jax.experimental import pallas as pl
from jax.experimental.pallas import tpu as pltpu

_K = 8
_TAU = 0.7
_T = 256


def _gating_kernel(x_ref, w_ref, disp_ref, comb_ref, loss_ref, carry_ref,
                   *, cap, G, nb, nj):
    i = pl.program_id(0)
    j = pl.program_id(1)

    @pl.when((i == 0) & (j == 0))
    def _init():
        carry_ref[...] = jnp.zeros_like(carry_ref)

    x = x_ref[0]
    w = w_ref[...]
    gates = jnp.dot(x, w, preferred_element_type=jnp.float32)
    gt = gates.T

    m = jnp.max(gt, axis=0, keepdims=True)
    ex = jnp.exp(gt - m)
    probs = ex / jnp.sum(ex, axis=0, keepdims=True)

    rowid = jax.lax.broadcasted_iota(jnp.int32, (G, _T), 0)

    work = probs
    csum = jnp.zeros((1, _T), jnp.float32)
    onehots = jnp.zeros((G, _T), jnp.float32)
    vmap_raw = jnp.zeros((G, _T), jnp.float32)
    renorm = jnp.zeros((1, _T), jnp.float32)
    for k in range(_K):
        mk = jnp.max(work, axis=0, keepdims=True)
        ismax = work == mk
        idx = jnp.min(jnp.where(ismax, rowid, G), axis=0, keepdims=True)
        oh = (rowid == idx).astype(jnp.float32)
        work = jnp.where(oh > 0.5, -1.0, work)
        csum = csum + mk
        if k == 0:
            keep = jnp.ones((1, _T), jnp.float32)
        else:
            keep = (csum < _TAU).astype(jnp.float32)
        onehots = onehots + oh * keep
        vmap_raw = vmap_raw + oh * (mk * keep)
        renorm = renorm + mk * keep
    v_map = vmap_raw / jnp.maximum(renorm, 1e-7)

    rowid_t = jax.lax.broadcasted_iota(jnp.int32, (_T, _T), 0)
    colid_t = jax.lax.broadcasted_iota(jnp.int32, (_T, _T), 1)
    ut = (rowid_t < colid_t).astype(jnp.float32)
    excl = jnp.dot(onehots, ut, preferred_element_type=jnp.float32)
    rank = excl + carry_ref[...]

    carry_ref[...] += jnp.sum(onehots, axis=1, keepdims=True)

    placed = (onehots > 0.5) & (rank < cap - 0.5)
    r_map = jnp.where(placed, rank, -1.0).astype(jnp.int32)

    ci = jax.lax.broadcasted_iota(jnp.int32, (G, cap, _T), 1)
    disp = (ci == r_map[:, None, :]).astype(jnp.float32)
    disp_ref[0] = disp
    comb_ref[0] = v_map[:, None, :] * disp

    @pl.when((i == nb - 1) & (j == nj - 1))
    def _loss():
        usage = jnp.minimum(carry_ref[...], float(cap))
        mu = jnp.mean(usage)
        l = jnp.mean((usage - mu) ** 2) / (mu + 1e-8)
        loss_ref[0, 0] = jnp.where(jnp.sum(usage) > 0, l, 0.0)


def kernel(x, W):
    b, n, d = x.shape
    G = W.shape[1]
    cap = max(min(n, int(n * 1.25 / G)), 4)
    nj = n // _T
    disp, comb, loss = pl.pallas_call(
        functools.partial(_gating_kernel, cap=cap, G=G, nb=b, nj=nj),
        grid=(b, nj),
        in_specs=[
            pl.BlockSpec((1, _T, d), lambda i, j: (i, j, 0)),
            pl.BlockSpec((d, G), lambda i, j: (0, 0)),
        ],
        out_specs=[
            pl.BlockSpec((1, G, cap, _T), lambda i, j: (i, 0, 0, j)),
            pl.BlockSpec((1, G, cap, _T), lambda i, j: (i, 0, 0, j)),
            pl.BlockSpec(memory_space=pltpu.SMEM),
        ],
        out_shape=[
            jax.ShapeDtypeStruct((b, G, cap, n), jnp.float32),
            jax.ShapeDtypeStruct((b, G, cap, n), jnp.float32),
            jax.ShapeDtypeStruct((1, 1), jnp.float32),
        ],
        scratch_shapes=[pltpu.VMEM((G, 1), jnp.float32)],
        compiler_params=pltpu.CompilerParams(
            dimension_semantics=("arbitrary", "arbitrary")),
    )(x, W)
    return (jnp.transpose(disp, (0, 3, 1, 2)),
            jnp.transpose(comb, (0, 3, 1, 2)),
            loss.reshape(()))


# speedup vs baseline: 7.9268x; 1.0921x over previous
import functools

import jax
import jax.numpy as jnp
from jax.experimental import pallas as pl
from jax.experimental.pallas import tpu as pltpu

_K = 8
_TAU = 0.7
_T = 512


def _gating_kernel(x_ref, w_ref, disp_ref, comb_ref, loss_ref, carry_ref,
                   *, cap, G, nb, nj):
    i = pl.program_id(0)
    j = pl.program_id(1)

    @pl.when((i == 0) & (j == 0))
    def _init():
        carry_ref[...] = jnp.zeros_like(carry_ref)

    x = x_ref[0]
    w = w_ref[...]
    gates = jnp.dot(x, w, preferred_element_type=jnp.float32)
    gt = gates.T

    m = jnp.max(gt, axis=0, keepdims=True)
    ex = jnp.exp(gt - m)
    probs = ex / jnp.sum(ex, axis=0, keepdims=True)

    rowid = jax.lax.broadcasted_iota(jnp.int32, (G, _T), 0)

    work = probs
    csum = jnp.zeros((1, _T), jnp.float32)
    onehots = jnp.zeros((G, _T), jnp.float32)
    vmap_raw = jnp.zeros((G, _T), jnp.float32)
    renorm = jnp.zeros((1, _T), jnp.float32)
    for k in range(_K):
        mk = jnp.max(work, axis=0, keepdims=True)
        ismax = work == mk
        idx = jnp.min(jnp.where(ismax, rowid, G), axis=0, keepdims=True)
        oh = (rowid == idx).astype(jnp.float32)
        work = jnp.where(oh > 0.5, -1.0, work)
        csum = csum + mk
        if k == 0:
            keep = jnp.ones((1, _T), jnp.float32)
        else:
            keep = (csum < _TAU).astype(jnp.float32)
        onehots = onehots + oh * keep
        vmap_raw = vmap_raw + oh * (mk * keep)
        renorm = renorm + mk * keep
    v_map = vmap_raw / jnp.maximum(renorm, 1e-7)

    rowid_t = jax.lax.broadcasted_iota(jnp.int32, (_T, _T), 0)
    colid_t = jax.lax.broadcasted_iota(jnp.int32, (_T, _T), 1)
    ut = (rowid_t < colid_t).astype(jnp.float32)
    excl = jnp.dot(onehots, ut, preferred_element_type=jnp.float32)
    rank = excl + carry_ref[...]

    carry_ref[...] += jnp.sum(onehots, axis=1, keepdims=True)

    placed = (onehots > 0.5) & (rank < cap - 0.5)
    r_map = jnp.where(placed, rank, -1.0).astype(jnp.int32)

    ci = jax.lax.broadcasted_iota(jnp.int32, (G, cap, _T), 1)
    disp = (ci == r_map[:, None, :]).astype(jnp.float32)
    disp_ref[0] = disp
    comb_ref[0] = v_map[:, None, :] * disp

    @pl.when((i == nb - 1) & (j == nj - 1))
    def _loss():
        usage = jnp.minimum(carry_ref[...], float(cap))
        mu = jnp.mean(usage)
        l = jnp.mean((usage - mu) ** 2) / (mu + 1e-8)
        loss_ref[0, 0] = jnp.where(jnp.sum(usage) > 0, l, 0.0)


def kernel(x, W):
    b, n, d = x.shape
    G = W.shape[1]
    cap = max(min(n, int(n * 1.25 / G)), 4)
    nj = n // _T
    disp, comb, loss = pl.pallas_call(
        functools.partial(_gating_kernel, cap=cap, G=G, nb=b, nj=nj),
        grid=(b, nj),
        in_specs=[
            pl.BlockSpec((1, _T, d), lambda i, j: (i, j, 0)),
            pl.BlockSpec((d, G), lambda i, j: (0, 0)),
        ],
        out_specs=[
            pl.BlockSpec((1, G, cap, _T), lambda i, j: (i, 0, 0, j)),
            pl.BlockSpec((1, G, cap, _T), lambda i, j: (i, 0, 0, j)),
            pl.BlockSpec(memory_space=pltpu.SMEM),
        ],
        out_shape=[
            jax.ShapeDtypeStruct((b, G, cap, n), jnp.float32),
            jax.ShapeDtypeStruct((b, G, cap, n), jnp.float32),
            jax.ShapeDtypeStruct((1, 1), jnp.float32),
        ],
        scratch_shapes=[pltpu.VMEM((G, 1), jnp.float32)],
        compiler_params=pltpu.CompilerParams(
            dimension_semantics=("arbitrary", "arbitrary")),
    )(x, W)
    return (jnp.transpose(disp, (0, 3, 1, 2)),
            jnp.transpose(comb, (0, 3, 1, 2)),
            loss.reshape(()))


# speedup vs baseline: 8.4522x; 1.0663x over previous
import functools

import jax
import jax.numpy as jnp
from jax.experimental import pallas as pl
from jax.experimental.pallas import tpu as pltpu

_K = 8
_TAU = 0.7
_T = 512


def _gating_kernel(x_ref, w_ref, disp_ref, comb_ref, loss_ref, carry_ref,
                   *, cap, G, nb, nj):
    i = pl.program_id(0)
    j = pl.program_id(1)

    @pl.when((i == 0) & (j == 0))
    def _init():
        carry_ref[...] = jnp.zeros_like(carry_ref)

    x = x_ref[0]
    wt = w_ref[...]
    gt = jax.lax.dot_general(wt, x, (((1,), (1,)), ((), ())),
                             preferred_element_type=jnp.float32)

    m = jnp.max(gt, axis=0, keepdims=True)
    ex = jnp.exp(gt - m)
    probs = ex / jnp.sum(ex, axis=0, keepdims=True)

    rowid = jax.lax.broadcasted_iota(jnp.int32, (G, _T), 0)

    work = probs
    csum = jnp.zeros((1, _T), jnp.float32)
    onehots = jnp.zeros((G, _T), jnp.float32)
    vmap_raw = jnp.zeros((G, _T), jnp.float32)
    renorm = jnp.zeros((1, _T), jnp.float32)
    for k in range(_K):
        mk = jnp.max(work, axis=0, keepdims=True)
        ismax = work == mk
        idx = jnp.min(jnp.where(ismax, rowid, G), axis=0, keepdims=True)
        oh = (rowid == idx).astype(jnp.float32)
        work = jnp.where(oh > 0.5, -1.0, work)
        csum = csum + mk
        if k == 0:
            keep = jnp.ones((1, _T), jnp.float32)
        else:
            keep = (csum < _TAU).astype(jnp.float32)
        onehots = onehots + oh * keep
        vmap_raw = vmap_raw + oh * (mk * keep)
        renorm = renorm + mk * keep
    v_map = vmap_raw / jnp.maximum(renorm, 1e-7)

    rowid_t = jax.lax.broadcasted_iota(jnp.int32, (_T, _T), 0)
    colid_t = jax.lax.broadcasted_iota(jnp.int32, (_T, _T), 1)
    ut = (rowid_t < colid_t).astype(jnp.float32)
    excl = jnp.dot(onehots, ut, preferred_element_type=jnp.float32)
    rank = excl + carry_ref[...]

    carry_ref[...] += jnp.sum(onehots, axis=1, keepdims=True)

    placed = (onehots > 0.5) & (rank < cap - 0.5)
    r_map = jnp.where(placed, rank, -1.0).astype(jnp.int32)

    ci = jax.lax.broadcasted_iota(jnp.int32, (G, cap, _T), 1)
    disp = (ci == r_map[:, None, :]).astype(jnp.float32)
    disp_ref[0] = disp
    comb_ref[0] = v_map[:, None, :] * disp

    @pl.when((i == nb - 1) & (j == nj - 1))
    def _loss():
        usage = jnp.minimum(carry_ref[...], float(cap))
        mu = jnp.mean(usage)
        l = jnp.mean((usage - mu) ** 2) / (mu + 1e-8)
        loss_ref[0, 0] = jnp.where(jnp.sum(usage) > 0, l, 0.0)


def kernel(x, W):
    b, n, d = x.shape
    G = W.shape[1]
    cap = max(min(n, int(n * 1.25 / G)), 4)
    nj = n // _T
    disp, comb, loss = pl.pallas_call(
        functools.partial(_gating_kernel, cap=cap, G=G, nb=b, nj=nj),
        grid=(b, nj),
        in_specs=[
            pl.BlockSpec((1, _T, d), lambda i, j: (i, j, 0)),
            pl.BlockSpec((G, d), lambda i, j: (0, 0)),
        ],
        out_specs=[
            pl.BlockSpec((1, G, cap, _T), lambda i, j: (i, 0, 0, j)),
            pl.BlockSpec((1, G, cap, _T), lambda i, j: (i, 0, 0, j)),
            pl.BlockSpec(memory_space=pltpu.SMEM),
        ],
        out_shape=[
            jax.ShapeDtypeStruct((b, G, cap, n), jnp.float32),
            jax.ShapeDtypeStruct((b, G, cap, n), jnp.float32),
            jax.ShapeDtypeStruct((1, 1), jnp.float32),
        ],
        scratch_shapes=[pltpu.VMEM((G, 1), jnp.float32)],
        compiler_params=pltpu.CompilerParams(
            dimension_semantics=("arbitrary", "arbitrary")),
    )(x, W.T)
    return (jnp.transpose(disp, (0, 3, 1, 2)),
            jnp.transpose(comb, (0, 3, 1, 2)),
            loss.reshape(()))
